# baseline (speedup 1.0000x reference)
import jax
import jax.numpy as jnp
from jax.experimental import pallas as pl
from jax.experimental.pallas import tpu as pltpu

_DIM = 256
_STATS_TILE = 100000
_MAIN_TILE = 10000


def _location(coors_ref, off_ref, win_ref):
    c = coors_ref[...].astype(jnp.float32)
    t = c - off_ref[...]
    t = t / win_ref[...]
    t = t * 2.0
    return t * 3.1415


def _h(coors_ref, off_ref, win_ref, w1_ref, b1_ref):
    l = _location(coors_ref, off_ref, win_ref).astype(jnp.bfloat16)
    return jnp.dot(l, w1_ref[...], preferred_element_type=jnp.float32) \
        + b1_ref[...]


def _stats_body(coorst_ref, offc_ref, winc_ref, w1f_ref, b1_ref,
                gamma_ref, mom_ref, mean_ref, scale_ref, *,
                nrows, nprog):
    i = pl.program_id(0)
    c = coorst_ref[...].astype(jnp.float32)
    t = c - offc_ref[...]
    t = t / winc_ref[...]
    t = t * 2.0
    t = t * 3.1415
    lb = t.astype(jnp.bfloat16).astype(jnp.float32)

    @pl.when(i == 0)
    def _init():
        mom_ref[...] = jnp.zeros_like(mom_ref)

    for j in range(3):
        mom_ref[3:4, j:j + 1] += jnp.sum(
            lb[:, j, :], axis=1, keepdims=True)
        for k in range(j, 3):
            p = jnp.sum(lb[:, j, :] * lb[:, k, :], axis=1, keepdims=True)
            mom_ref[j:j + 1, k:k + 1] += p

    @pl.when(i == nprog - 1)
    def _finish():
        inv_n = 1.0 / nrows
        m = [mom_ref[3, j] * inv_n for j in range(3)]
        mean = b1_ref[...]
        var = jnp.zeros((1, _DIM), jnp.float32)
        for j in range(3):
            mean = mean + m[j] * w1f_ref[j:j + 1, :]
            for k in range(j, 3):
                cjk = mom_ref[j, k] * inv_n - m[j] * m[k]
                w = 1.0 if j == k else 2.0
                var = var + (w * cjk) * (w1f_ref[j:j + 1, :]
                                         * w1f_ref[k:k + 1, :])
        mean_ref[...] = mean
        scale_ref[...] = gamma_ref[...] / jnp.sqrt(var + 1e-5)


def _main_body(coors_ref, feat_ref, off_ref, win_ref, w1_ref, b1_ref,
               mean_ref, scale_ref, beta_ref, w2_ref, b2_ref, out_ref):
    h = _h(coors_ref, off_ref, win_ref, w1_ref, b1_ref)
    hn = (h - mean_ref[...]) * scale_ref[...] + beta_ref[...]
    u = jnp.maximum(hn, 0.0).astype(jnp.bfloat16)
    acc = jnp.dot(u, w2_ref[...], preferred_element_type=jnp.float32)
    out_ref[...] = acc + b2_ref[...] + feat_ref[...]


def kernel(features, coors, W1, b1, gamma, beta, W2, b2):
    n = features.shape[0]
    nf = jnp.float32(n)

    w1r = W1[::-1].astype(jnp.bfloat16)
    off = jnp.array([[20.5, 720.0, 720.0]], dtype=jnp.float32)
    win = jnp.array([[41.0, 1440.0, 1440.0]], dtype=jnp.float32)
    b1r = b1[None, :]

    common_specs = [
        pl.BlockSpec((1, 3), lambda i: (0, 0)),
        pl.BlockSpec((1, 3), lambda i: (0, 0)),
        pl.BlockSpec((3, _DIM), lambda i: (0, 0)),
        pl.BlockSpec((1, _DIM), lambda i: (0, 0)),
    ]

    import functools as _ft
    nchunk = n // _STATS_TILE
    coorst = coors.reshape(nchunk, _STATS_TILE, 3).transpose(0, 2, 1)
    offc = off.reshape(1, 3, 1)
    winc = win.reshape(1, 3, 1)
    w1f = w1r.astype(jnp.float32)
    vec_spec = pl.BlockSpec((1, _DIM), lambda i: (0, 0))
    _, mean, scale = pl.pallas_call(
        _ft.partial(_stats_body, nrows=float(n), nprog=nchunk),
        grid=(nchunk,),
        in_specs=[
            pl.BlockSpec((1, 3, _STATS_TILE), lambda i: (i, 0, 0)),
            pl.BlockSpec((1, 3, 1), lambda i: (0, 0, 0)),
            pl.BlockSpec((1, 3, 1), lambda i: (0, 0, 0)),
            pl.BlockSpec((3, _DIM), lambda i: (0, 0)),
            vec_spec,
            vec_spec,
        ],
        out_specs=[
            pl.BlockSpec((4, 3), lambda i: (0, 0)),
            vec_spec,
            vec_spec,
        ],
        out_shape=[
            jax.ShapeDtypeStruct((4, 3), jnp.float32),
            jax.ShapeDtypeStruct((1, _DIM), jnp.float32),
            jax.ShapeDtypeStruct((1, _DIM), jnp.float32),
        ],
        compiler_params=pltpu.CompilerParams(
            dimension_semantics=("arbitrary",)),
    )(coorst, offc, winc, w1f, b1r, gamma[None, :])

    out = pl.pallas_call(
        _main_body,
        grid=(n // _MAIN_TILE,),
        in_specs=[
            pl.BlockSpec((_MAIN_TILE, 3), lambda i: (i, 0)),
            pl.BlockSpec((_MAIN_TILE, _DIM), lambda i: (i, 0)),
        ]
        + common_specs
        + [
            pl.BlockSpec((1, _DIM), lambda i: (0, 0)),
            pl.BlockSpec((1, _DIM), lambda i: (0, 0)),
            pl.BlockSpec((1, _DIM), lambda i: (0, 0)),
            pl.BlockSpec((_DIM, _DIM), lambda i: (0, 0)),
            pl.BlockSpec((1, _DIM), lambda i: (0, 0)),
        ],
        out_specs=pl.BlockSpec((_MAIN_TILE, _DIM), lambda i: (i, 0)),
        out_shape=jax.ShapeDtypeStruct((n, _DIM), jnp.float32),
        compiler_params=pltpu.CompilerParams(
            dimension_semantics=("parallel",)),
    )(coors, features, off, win, w1r, b1r, mean, scale, beta[None, :],
      W2.astype(jnp.bfloat16), b2[None, :])
    return out


# speedup vs baseline: 1.0038x; 1.0038x over previous
import jax
import jax.numpy as jnp
from jax.experimental import pallas as pl
from jax.experimental.pallas import tpu as pltpu

_DIM = 256
_STATS_TILE = 40000
_MAIN_TILE = 10000


def _location(coors_ref, off_ref, win_ref):
    c = coors_ref[...].astype(jnp.float32)
    t = c - off_ref[...]
    t = t / win_ref[...]
    t = t * 2.0
    return t * 3.1415


def _h(coors_ref, off_ref, win_ref, w1_ref, b1_ref):
    l = _location(coors_ref, off_ref, win_ref).astype(jnp.bfloat16)
    return jnp.dot(l, w1_ref[...], preferred_element_type=jnp.float32) \
        + b1_ref[...]


def _stats_body(coorst_ref, offc_ref, winc_ref, w1f_ref, b1_ref,
                gamma_ref, mom_ref, mean_ref, scale_ref, *,
                nrows, nprog):
    i = pl.program_id(0)
    c = coorst_ref[...].astype(jnp.float32)
    t = c - offc_ref[...]
    t = t / winc_ref[...]
    t = t * 2.0
    t = t * 3.1415
    lb = t.astype(jnp.bfloat16).astype(jnp.float32)

    @pl.when(i == 0)
    def _init():
        mom_ref[...] = jnp.zeros_like(mom_ref)

    for j in range(3):
        mom_ref[3:4, j:j + 1] += jnp.sum(
            lb[:, j, :], axis=1, keepdims=True)
        for k in range(j, 3):
            p = jnp.sum(lb[:, j, :] * lb[:, k, :], axis=1, keepdims=True)
            mom_ref[j:j + 1, k:k + 1] += p

    @pl.when(i == nprog - 1)
    def _finish():
        inv_n = 1.0 / nrows
        m = [mom_ref[3, j] * inv_n for j in range(3)]
        mean = b1_ref[...]
        var = jnp.zeros((1, _DIM), jnp.float32)
        for j in range(3):
            mean = mean + m[j] * w1f_ref[j:j + 1, :]
            for k in range(j, 3):
                cjk = mom_ref[j, k] * inv_n - m[j] * m[k]
                w = 1.0 if j == k else 2.0
                var = var + (w * cjk) * (w1f_ref[j:j + 1, :]
                                         * w1f_ref[k:k + 1, :])
        mean_ref[...] = mean
        scale_ref[...] = gamma_ref[...] / jnp.sqrt(var + 1e-5)


def _main_body(coors_ref, feat_ref, off_ref, win_ref, w1_ref, b1_ref,
               mean_ref, scale_ref, beta_ref, w2_ref, b2_ref, out_ref):
    h = _h(coors_ref, off_ref, win_ref, w1_ref, b1_ref)
    hn = (h - mean_ref[...]) * scale_ref[...] + beta_ref[...]
    u = jnp.maximum(hn, 0.0).astype(jnp.bfloat16)
    acc = jnp.dot(u, w2_ref[...], preferred_element_type=jnp.float32)
    out_ref[...] = acc + b2_ref[...] + feat_ref[...]


def kernel(features, coors, W1, b1, gamma, beta, W2, b2):
    n = features.shape[0]
    nf = jnp.float32(n)

    w1r = W1[::-1].astype(jnp.bfloat16)
    off = jnp.array([[20.5, 720.0, 720.0]], dtype=jnp.float32)
    win = jnp.array([[41.0, 1440.0, 1440.0]], dtype=jnp.float32)
    b1r = b1[None, :]

    common_specs = [
        pl.BlockSpec((1, 3), lambda i: (0, 0)),
        pl.BlockSpec((1, 3), lambda i: (0, 0)),
        pl.BlockSpec((3, _DIM), lambda i: (0, 0)),
        pl.BlockSpec((1, _DIM), lambda i: (0, 0)),
    ]

    import functools as _ft
    nchunk = n // _STATS_TILE
    coorst = coors.reshape(nchunk, _STATS_TILE, 3).transpose(0, 2, 1)
    offc = off.reshape(1, 3, 1)
    winc = win.reshape(1, 3, 1)
    w1f = w1r.astype(jnp.float32)
    vec_spec = pl.BlockSpec((1, _DIM), lambda i: (0, 0))
    _, mean, scale = pl.pallas_call(
        _ft.partial(_stats_body, nrows=float(n), nprog=nchunk),
        grid=(nchunk,),
        in_specs=[
            pl.BlockSpec((1, 3, _STATS_TILE), lambda i: (i, 0, 0)),
            pl.BlockSpec((1, 3, 1), lambda i: (0, 0, 0)),
            pl.BlockSpec((1, 3, 1), lambda i: (0, 0, 0)),
            pl.BlockSpec((3, _DIM), lambda i: (0, 0)),
            vec_spec,
            vec_spec,
        ],
        out_specs=[
            pl.BlockSpec((4, 3), lambda i: (0, 0)),
            vec_spec,
            vec_spec,
        ],
        out_shape=[
            jax.ShapeDtypeStruct((4, 3), jnp.float32),
            jax.ShapeDtypeStruct((1, _DIM), jnp.float32),
            jax.ShapeDtypeStruct((1, _DIM), jnp.float32),
        ],
        compiler_params=pltpu.CompilerParams(
            dimension_semantics=("arbitrary",)),
    )(coorst, offc, winc, w1f, b1r, gamma[None, :])

    out = pl.pallas_call(
        _main_body,
        grid=(n // _MAIN_TILE,),
        in_specs=[
            pl.BlockSpec((_MAIN_TILE, 3), lambda i: (i, 0)),
            pl.BlockSpec((_MAIN_TILE, _DIM), lambda i: (i, 0)),
        ]
        + common_specs
        + [
            pl.BlockSpec((1, _DIM), lambda i: (0, 0)),
            pl.BlockSpec((1, _DIM), lambda i: (0, 0)),
            pl.BlockSpec((1, _DIM), lambda i: (0, 0)),
            pl.BlockSpec((_DIM, _DIM), lambda i: (0, 0)),
            pl.BlockSpec((1, _DIM), lambda i: (0, 0)),
        ],
        out_specs=pl.BlockSpec((_MAIN_TILE, _DIM), lambda i: (i, 0)),
        out_shape=jax.ShapeDtypeStruct((n, _DIM), jnp.float32),
        compiler_params=pltpu.CompilerParams(
            dimension_semantics=("parallel",)),
    )(coors, features, off, win, w1r, b1r, mean, scale, beta[None, :],
      W2.astype(jnp.bfloat16), b2[None, :])
    return out
